# baseline (speedup 1.0000x reference)
import functools

import jax
import jax.numpy as jnp
from jax import lax
from jax.experimental import pallas as pl
from jax.experimental.pallas import tpu as pltpu
from jax.experimental.pallas import tpu_sc as plsc

N = 10000
E = 320000
G = 128
NT = 16
ET = 5
ND = 128
ED = 16
NG = 16
OUT = 1

NPAD = 10240
CHN = 2048
GN = NPAD // CHN
CHE = 3200
GE = E // CHE
NGRP = E // 128
NWORK = 32
GPW = (NGRP + NWORK - 1) // NWORK
GPS = (NGRP + 15) // 16
HALF = NPAD // 2
RPS = HALF // 16
SPAD = HALF + 8

_SMEAR_STEP = 10.0 / (NG - 1)
_SMEAR_COEFF = -0.5 / _SMEAR_STEP ** 2

_pc = pl.pallas_call
_f32 = jnp.float32


def _mm(a, b):
    return jnp.dot(a, b, preferred_element_type=_f32)


def _bcast_spec(shape):
    return pl.BlockSpec(shape, lambda i: (0,) * len(shape))


def _row_spec(rows, cols):
    return pl.BlockSpec((rows, cols), lambda i: (i, 0))


def _onehot(b_ref):
    b = b_ref[0, 0, :]
    ids = jnp.reshape(b, (b.shape[0], 1))
    return (ids == lax.broadcasted_iota(jnp.int32, (b.shape[0], G), 1)).astype(_f32)


def _tc_pre_body(hn_ref, wemb_ref, h0_ref):
    h0_ref[...] = _mm(hn_ref[...], wemb_ref[...])


def _tc_pre(hnp, wemb):
    return _pc(
        _tc_pre_body,
        grid=(GN,),
        in_specs=[_row_spec(CHN, NT), _bcast_spec((NT, ND))],
        out_specs=_row_spec(CHN, ND),
        out_shape=jax.ShapeDtypeStruct((NPAD, ND), _f32),
    )(hnp, wemb)


def _edge_core(he, hs, hd, dfeat, te, w):
    pre_e = (_mm(he, w['w1he']) + _mm(hs, w['w1s']) + _mm(hd, w['w1d'])
             + _mm(dfeat, w['w1df']) + te * w['w1t'] + w['eb1'])
    he_new = he + _mm(jnp.maximum(pre_e, 0.0), w['ew2']) + w['eb2']
    pre_m = (_mm(hs, w['m1h']) + _mm(he_new, w['m1e'])
             + _mm(dfeat, w['m1df']) + w['mb1'])
    return he_new, jnp.maximum(pre_m, 0.0)


def _edge_wdict(refs):
    names = ('w1he', 'w1s', 'w1d', 'w1df', 'w1t', 'eb1', 'ew2', 'eb2',
             'm1h', 'm1e', 'm1df', 'mb1')
    return {k: r[...] for k, r in zip(names, refs)}


_EDGE_W_SPECS = [
    _bcast_spec((ED, ED)), _bcast_spec((ND, ED)), _bcast_spec((ND, ED)),
    _bcast_spec((NG, ED)), _bcast_spec((1, ED)), _bcast_spec((1, ED)),
    _bcast_spec((ED, ED)), _bcast_spec((1, ED)),
    _bcast_spec((ND, ND)), _bcast_spec((ED, ND)), _bcast_spec((NG, ND)),
    _bcast_spec((1, ND)),
]


def _tc_edge1_body(hein_ref, hs_ref, hd_ref, ps_ref, pd_ref, be_ref, tcol_ref,
                   wemb_ref, *rest):
    wrefs, (he_ref, r_ref, df_ref) = rest[:12], rest[12:]
    he = _mm(hein_ref[...], wemb_ref[...])
    rel = pd_ref[:, 0:3] - ps_ref[:, 0:3]
    dist = jnp.sqrt(jnp.sum(rel * rel, axis=1, keepdims=True) + 1e-8)
    offs = lax.broadcasted_iota(jnp.int32, (1, NG), 1).astype(_f32) * _SMEAR_STEP
    dfeat = jnp.exp(_SMEAR_COEFF * (dist - offs) ** 2)
    df_ref[...] = dfeat
    te = _mm(_onehot(be_ref), tcol_ref[...])
    he_new, r = _edge_core(he, hs_ref[...], hd_ref[...], dfeat, te,
                           _edge_wdict(wrefs))
    he_ref[...] = he_new
    r_ref[...] = r


def _tc_edge1(hein, hs, hd, psg, pdg, be3, tcol, wemb, wts):
    return _pc(
        _tc_edge1_body,
        grid=(GE,),
        in_specs=[
            _row_spec(CHE, ET), _row_spec(CHE, ND), _row_spec(CHE, ND),
            _row_spec(CHE, ND), _row_spec(CHE, ND),
            pl.BlockSpec((1, 1, CHE), lambda i: (i, 0, 0)),
            _bcast_spec((G, 1)), _bcast_spec((ET, ED)),
            *_EDGE_W_SPECS,
        ],
        out_specs=[_row_spec(CHE, ED), _row_spec(CHE, ND), _row_spec(CHE, NG)],
        out_shape=[
            jax.ShapeDtypeStruct((E, ED), _f32),
            jax.ShapeDtypeStruct((E, ND), _f32),
            jax.ShapeDtypeStruct((E, NG), _f32),
        ],
    )(hein, hs, hd, psg, pdg, be3, tcol, wemb, *wts)


def _tc_edge23_body(last, he_in_ref, hs_ref, hd_ref, df_ref, be_ref, tcol_ref,
                    *rest):
    if last:
        wrefs, (he_ref, r_ref, esum_ref, ecnt_ref) = rest[:12], rest[12:]
    else:
        wrefs, (he_ref, r_ref) = rest[:12], rest[12:]
    oh = _onehot(be_ref)
    te = _mm(oh, tcol_ref[...])
    he_new, r = _edge_core(he_in_ref[...], hs_ref[...], hd_ref[...],
                           df_ref[...], te, _edge_wdict(wrefs))
    he_ref[...] = he_new
    r_ref[...] = r
    if last:
        @pl.when(pl.program_id(0) == 0)
        def _():
            esum_ref[...] = jnp.zeros_like(esum_ref)
            ecnt_ref[...] = jnp.zeros_like(ecnt_ref)

        esum_ref[...] += lax.dot_general(oh, he_new, (((0,), (0,)), ((), ())),
                                         preferred_element_type=_f32)
        ecnt_ref[...] += jnp.sum(oh, axis=0, keepdims=True)


def _tc_edge23(last, he, hs, hd, dfeat, be3, tcol, wts):
    out_specs = [_row_spec(CHE, ED), _row_spec(CHE, ND)]
    out_shape = [jax.ShapeDtypeStruct((E, ED), _f32),
                 jax.ShapeDtypeStruct((E, ND), _f32)]
    if last:
        out_specs += [_bcast_spec((G, ED)), _bcast_spec((1, G))]
        out_shape += [jax.ShapeDtypeStruct((G, ED), _f32),
                      jax.ShapeDtypeStruct((1, G), _f32)]
    return _pc(
        functools.partial(_tc_edge23_body, last),
        grid=(GE,),
        in_specs=[
            _row_spec(CHE, ED), _row_spec(CHE, ND), _row_spec(CHE, ND),
            _row_spec(CHE, NG),
            pl.BlockSpec((1, 1, CHE), lambda i: (i, 0, 0)),
            _bcast_spec((G, 1)),
            *_EDGE_W_SPECS,
        ],
        out_specs=out_specs,
        out_shape=out_shape,
    )(he, hs, hd, dfeat, be3, tcol, *wts)


def _tc_node_body(last, hn_ref, s_ref, d_ref, bn_ref,
                  tcol_ref, mw2_ref, mb2_ref, n1h_ref, n1a_ref, n1t_ref,
                  nb1_ref, nw2_ref, nb2_ref, *rest):
    oh = _onehot(bn_ref)
    nt = _mm(oh, tcol_ref[...])
    deg = d_ref[:, 0:1]
    agg = _mm(s_ref[...], mw2_ref[...]) + deg * mb2_ref[...]
    pre_n = (_mm(hn_ref[...], n1h_ref[...]) + _mm(agg, n1a_ref[...])
             + nt * n1t_ref[...] + nb1_ref[...])
    hn_new = (hn_ref[...] + _mm(jnp.maximum(pre_n, 0.0), nw2_ref[...])
              + nb2_ref[...])
    if last:
        hn_out, nsum_ref, ncnt_ref = rest
        hn_out[...] = hn_new

        @pl.when(pl.program_id(0) == 0)
        def _():
            nsum_ref[...] = jnp.zeros_like(nsum_ref)
            ncnt_ref[...] = jnp.zeros_like(ncnt_ref)

        nsum_ref[...] += lax.dot_general(oh, hn_new, (((0,), (0,)), ((), ())),
                                         preferred_element_type=_f32)
        ncnt_ref[...] += jnp.sum(oh, axis=0, keepdims=True)
    else:
        rest[0][...] = hn_new


def _tc_node(last, hn, sacc, deg, bn3, tcol, wts):
    in_specs = [
        _row_spec(CHN, ND), _row_spec(CHN, ND), _row_spec(CHN, ND),
        pl.BlockSpec((1, 1, CHN), lambda i: (i, 0, 0)),
        _bcast_spec((G, 1)),
        _bcast_spec((ND, ND)), _bcast_spec((1, ND)),
        _bcast_spec((ND, ND)), _bcast_spec((ND, ND)), _bcast_spec((1, ND)),
        _bcast_spec((1, ND)), _bcast_spec((ND, ND)), _bcast_spec((1, ND)),
    ]
    if last:
        out_specs = [_row_spec(CHN, ND), _bcast_spec((G, ND)),
                     _bcast_spec((1, G))]
        out_shape = [jax.ShapeDtypeStruct((NPAD, ND), _f32),
                     jax.ShapeDtypeStruct((G, ND), _f32),
                     jax.ShapeDtypeStruct((1, G), _f32)]
    else:
        out_specs = [_row_spec(CHN, ND)]
        out_shape = [jax.ShapeDtypeStruct((NPAD, ND), _f32)]
    return _pc(
        functools.partial(_tc_node_body, last),
        grid=(GN,),
        in_specs=in_specs,
        out_specs=out_specs,
        out_shape=out_shape,
    )(hn, sacc, deg, bn3, tcol, *wts)


def _tc_final_body(nsum_ref, ncnt_ref, esum_ref, ecnt_ref,
                   fw1_ref, fb1_ref, fw2_ref, fb2_ref, out_ref):
    cn = jnp.maximum(jnp.transpose(ncnt_ref[...]), 1.0)
    ce = jnp.maximum(jnp.transpose(ecnt_ref[...]), 1.0)
    hsub = jnp.concatenate([nsum_ref[...] / cn, esum_ref[...] / ce], axis=1)
    h = jnp.maximum(_mm(hsub, fw1_ref[...]) + fb1_ref[...], 0.0)
    out_ref[...] = _mm(h, fw2_ref[...]) + fb2_ref[...]


def _tc_final(nsum, ncnt, esum, ecnt, fw1, fb1, fw2, fb2):
    return _pc(
        _tc_final_body,
        out_shape=jax.ShapeDtypeStruct((G, OUT), _f32),
    )(nsum, ncnt, esum, ecnt, fw1, fb1, fw2, fb2)


@functools.lru_cache(maxsize=1)
def _mesh():
    return plsc.VectorSubcoreMesh(core_axis_name="c", subcore_axis_name="s")


def _range32(w):
    per = NGRP // NWORK
    lo = w * per + jnp.minimum(w, NGRP % NWORK)
    hi = lo + per + (w < NGRP % NWORK).astype(jnp.int32)
    return lo, hi


def _range16(s):
    per = NGRP // 16
    lo = s * per + jnp.minimum(s, NGRP % 16)
    hi = lo + per + (s < NGRP % 16).astype(jnp.int32)
    return lo, hi


NSLOT = 3
NIT32 = (NGRP // NWORK + 1 + NSLOT - 1) // NSLOT
NIT16 = (NGRP // 16 + 1 + NSLOT - 1) // NSLOT
IDXROWS32 = 96
IDXROWS16 = 168
NGRPP = 2560


def _aligned_base(lo):
    base = pl.multiple_of((lo // 8) * 8, 8)
    return base, lo - base


def _drain(dummy_hbm, buf, sem):
    pltpu.make_async_copy(dummy_hbm.at[pl.ds(0, 128)], buf, sem).wait()


def _remap_to_half(idxrow, base):
    for kk in range(8):
        sl = pl.ds(kk * 16, 16)
        v = idxrow[sl] - base
        oob = (v < 0) | (v >= HALF)
        idxrow[sl] = jnp.where(oob, HALF, v)


def _gather_ring(tab_h, gs_h, gd_h, sall, dall, bufs, bufd, gsems, wsems,
                 lo, hi, off):
    def body(j, carry):
        for s in range(NSLOT):
            k = NSLOT * j + s + off
            g = lo + NSLOT * j + s

            @pl.when((j > 0) & (g - NSLOT < hi))
            def _():
                _drain(tab_h, bufs[s], wsems[s])
                _drain(tab_h, bufd[s], wsems[s])

            @pl.when(g < hi)
            def _():
                pltpu.async_copy(tab_h.at[sall.at[k]], bufs[s], gsems[s])
                pltpu.async_copy(tab_h.at[dall.at[k]], bufd[s], gsems[s])

        for s in range(NSLOT):
            k = NSLOT * j + s + off
            g = lo + NSLOT * j + s

            @pl.when(g < hi)
            def _():
                _drain(tab_h, bufs[s], gsems[s])
                _drain(tab_h, bufd[s], gsems[s])
                rows = pl.ds(g * 128, 128)
                pltpu.async_copy(bufs[s], gs_h.at[rows], wsems[s])
                pltpu.async_copy(bufd[s], gd_h.at[rows], wsems[s])

        return carry

    lax.fori_loop(0, NIT32, body, 0)
    for s in range(NSLOT):
        g = lo + NSLOT * (NIT32 - 1) + s

        @pl.when(g < hi)
        def _():
            _drain(tab_h, bufs[s], wsems[s])
            _drain(tab_h, bufd[s], wsems[s])


_GATHER_SCRATCH = [
    pltpu.VMEM((IDXROWS32, 128), jnp.int32),
    pltpu.VMEM((IDXROWS32, 128), jnp.int32),
    pltpu.VMEM((NSLOT, 128, ND), _f32),
    pltpu.VMEM((NSLOT, 128, ND), _f32),
] + [pltpu.SemaphoreType.DMA] * (2 * NSLOT)


def _sc_gather_h(h, src2, dst2):
    @functools.partial(
        pl.kernel, mesh=_mesh(),
        out_type=[jax.ShapeDtypeStruct((E, ND), _f32),
                  jax.ShapeDtypeStruct((E, ND), _f32)],
        scratch_types=_GATHER_SCRATCH,
    )
    def k(h_h, src_h, dst_h, hs_h, hd_h, sall, dall, bufs, bufd, *sems):
        gsems, wsems = sems[:NSLOT], sems[NSLOT:]
        w = lax.axis_index("s") * 2 + lax.axis_index("c")
        lo, hi = _range32(w)
        p8, off = _aligned_base(lo)
        pltpu.sync_copy(src_h.at[pl.ds(p8, IDXROWS32)], sall)
        pltpu.sync_copy(dst_h.at[pl.ds(p8, IDXROWS32)], dall)
        bs = [bufs.at[s] for s in range(NSLOT)]
        bd = [bufd.at[s] for s in range(NSLOT)]
        _gather_ring(h_h, hs_h, hd_h, sall, dall, bs, bd, gsems, wsems,
                     lo, hi, off)

    return k(h, src2, dst2)


def _sc_deg(dst2, znd, ones):
    @functools.partial(
        pl.kernel, mesh=_mesh(),
        out_type=jax.ShapeDtypeStruct((NPAD, ND), _f32),
        scratch_types=[pltpu.VMEM((IDXROWS16, 128), jnp.int32),
                       pltpu.VMEM((128, ND), _f32),
                       pltpu.VMEM_SHARED((SPAD, ND), _f32),
                       pltpu.SemaphoreType.DMA],
    )
    def k(dst_h, znd_h, ones_h, deg_h, ddeg, obuf, deg_sh, asem):
        cid = lax.axis_index("c")
        s = lax.axis_index("s")
        base = cid * HALF
        lo, hi = _range16(s)
        p8, off = _aligned_base(lo)
        my = pl.ds(s * RPS, RPS)
        pltpu.sync_copy(znd_h.at[my], deg_sh.at[my])
        pltpu.sync_copy(ones_h, obuf)
        pltpu.sync_copy(dst_h.at[pl.ds(p8, IDXROWS16)], ddeg)
        plsc.subcore_barrier()

        def dbody(j, carry):
            _remap_to_half(ddeg.at[j + off], base)
            pltpu.async_copy(obuf, deg_sh.at[ddeg.at[j + off]], asem,
                             add=True)
            return carry

        lax.fori_loop(0, hi - lo, dbody, 0)

        def wbody(j, carry):
            _drain(znd_h, obuf, asem)
            return carry

        lax.fori_loop(0, hi - lo, wbody, 0)
        plsc.subcore_barrier()
        pltpu.sync_copy(deg_sh.at[my], deg_h.at[pl.ds(base + s * RPS, RPS)])

    return k(dst2, znd, ones)


def _sc_scatter(r, dst2, znd):
    @functools.partial(
        pl.kernel, mesh=_mesh(),
        out_type=jax.ShapeDtypeStruct((NPAD, ND), _f32),
        scratch_types=[pltpu.VMEM((IDXROWS16, 128), jnp.int32),
                       pltpu.VMEM((NSLOT, 128, ND), _f32),
                       pltpu.VMEM_SHARED((SPAD, ND), _f32)]
        + [pltpu.SemaphoreType.DMA] * (2 * NSLOT),
    )
    def k(r_h, dst_h, znd_h, s_h, dall, rbuf, s_sh, *sems):
        rsems, asems = sems[:NSLOT], sems[NSLOT:]
        cid = lax.axis_index("c")
        s = lax.axis_index("s")
        base = cid * HALF
        lo, hi = _range16(s)
        p8, off = _aligned_base(lo)
        my = pl.ds(s * RPS, RPS)
        pltpu.sync_copy(znd_h.at[my], s_sh.at[my])
        pltpu.sync_copy(dst_h.at[pl.ds(p8, IDXROWS16)], dall)
        plsc.subcore_barrier()

        @pl.when(lo < hi)
        def _():
            pltpu.async_copy(r_h.at[pl.ds(lo * 128, 128)], rbuf.at[0],
                             rsems[0])

        def body(j, carry):
            for sl in range(NSLOT):
                k_ = NSLOT * j + sl
                g = lo + k_
                bp = (sl + 1) % NSLOT
                kn = k_ + 1
                gn = lo + kn

                @pl.when(gn < hi)
                def _():
                    @pl.when(kn >= NSLOT)
                    def _():
                        _drain(r_h, rbuf.at[bp], asems[bp])

                    pltpu.async_copy(r_h.at[pl.ds(gn * 128, 128)],
                                     rbuf.at[bp], rsems[bp])

                @pl.when(g < hi)
                def _():
                    _drain(r_h, rbuf.at[sl], rsems[sl])
                    _remap_to_half(dall.at[k_ + off], base)
                    pltpu.async_copy(rbuf.at[sl], s_sh.at[dall.at[k_ + off]],
                                     asems[sl], add=True)

            return carry

        lax.fori_loop(0, NIT16, body, 0)
        for sl in range(NSLOT):
            @pl.when(lo + sl < hi)
            def _():
                _drain(r_h, rbuf.at[sl], asems[sl])

        plsc.subcore_barrier()
        pltpu.sync_copy(s_sh.at[my], s_h.at[pl.ds(base + s * RPS, RPS)])

    return k(r, dst2, znd)


def _block_weights(blk):
    w1 = blk['edge_W1']
    m1 = blk['msg_W1']
    nw1 = blk['node_W1']
    edge = (w1[:ED], w1[ED:ED + ND], w1[ED + ND:ED + 2 * ND],
            w1[ED + 2 * ND:ED + 2 * ND + NG],
            w1[ED + 2 * ND + NG:].reshape(1, ED),
            blk['edge_b1'].reshape(1, ED), blk['edge_W2'],
            blk['edge_b2'].reshape(1, ED),
            m1[:ND], m1[ND:ND + ED], m1[ND + ED:],
            blk['msg_b1'].reshape(1, ND))
    node = (blk['msg_W2'], blk['msg_b2'].reshape(1, ND),
            nw1[:ND], nw1[ND:2 * ND], nw1[2 * ND:].reshape(1, ND),
            blk['node_b1'].reshape(1, ND), blk['node_W2'],
            blk['node_b2'].reshape(1, ND))
    return edge, node


def kernel(h_node, pos_node, batch_node, h_edge, edge_index, batch_edge, t,
           params):
    blocks = params['blocks']
    hnp = jnp.pad(h_node, ((0, NPAD - N), (0, 0)))
    posp = jnp.pad(pos_node, ((0, NPAD - N), (0, ND - 3)))
    bn3 = jnp.pad(batch_node, (0, NPAD - N), constant_values=G).reshape(
        GN, 1, CHN)
    be3 = batch_edge.reshape(GE, 1, CHE)
    src2 = jnp.pad(edge_index[0].reshape(NGRP, 128),
                   ((0, NGRPP - NGRP), (0, 0)))
    dst2 = jnp.pad(edge_index[1].reshape(NGRP, 128),
                   ((0, NGRPP - NGRP), (0, 0)))
    tcol = t.astype(_f32).reshape(G, 1)
    znd = jnp.zeros((NPAD, ND), _f32)
    ones = jnp.ones((128, ND), _f32)

    ew, nw = zip(*(_block_weights(b) for b in blocks))

    h0 = _tc_pre(hnp, params['W_node_emb'])
    psg, pdg = _sc_gather_h(posp, src2, dst2)
    deg = _sc_deg(dst2, znd, ones)
    hs, hd = _sc_gather_h(h0, src2, dst2)
    he, r, dfeat = _tc_edge1(h_edge, hs, hd, psg, pdg, be3, tcol,
                             params['W_edge_emb'], ew[0])
    sacc = _sc_scatter(r, dst2, znd)
    hn = _tc_node(False, h0, sacc, deg, bn3, tcol, nw[0])[0]

    hs, hd = _sc_gather_h(hn, src2, dst2)
    he, r = _tc_edge23(False, he, hs, hd, dfeat, be3, tcol, ew[1])
    sacc = _sc_scatter(r, dst2, znd)
    hn = _tc_node(False, hn, sacc, deg, bn3, tcol, nw[1])[0]

    hs, hd = _sc_gather_h(hn, src2, dst2)
    he, r, esum, ecnt = _tc_edge23(True, he, hs, hd, dfeat, be3, tcol, ew[2])
    sacc = _sc_scatter(r, dst2, znd)
    hn, nsum, ncnt = _tc_node(True, hn, sacc, deg, bn3, tcol, nw[2])

    return _tc_final(nsum, ncnt, esum, ecnt, params['final_W1'],
                     params['final_b1'].reshape(1, ND + ED),
                     params['final_W2'], params['final_b2'].reshape(1, OUT))


# speedup vs baseline: 1.0350x; 1.0350x over previous
import functools

import jax
import jax.numpy as jnp
from jax import lax
from jax.experimental import pallas as pl
from jax.experimental.pallas import tpu as pltpu
from jax.experimental.pallas import tpu_sc as plsc

N = 10000
E = 320000
G = 128
NT = 16
ET = 5
ND = 128
ED = 16
NG = 16
OUT = 1

NPAD = 10240
CHN = 2048
GN = NPAD // CHN
CHE = 3200
GE = E // CHE
NGRP = E // 128
NWORK = 32
HALF = NPAD // 2
RPS = HALF // 16
SPAD = HALF + 8

NSLOT = 3
NIT32 = (NGRP // NWORK + 1 + NSLOT - 1) // NSLOT
NIT16 = (NGRP // 16 + 1 + NSLOT - 1) // NSLOT
IDXROWS32 = 96
IDXROWS16 = 168
NGRPP = 2560

_SMEAR_STEP = 10.0 / (NG - 1)
_SMEAR_COEFF = -0.5 / _SMEAR_STEP ** 2

_pc = pl.pallas_call
_f32 = jnp.float32


def _mm(a, b):
    return jnp.dot(a, b, preferred_element_type=_f32)


def _bcast_spec(shape):
    return pl.BlockSpec(shape, lambda i: (0,) * len(shape))


def _row_spec(rows, cols):
    return pl.BlockSpec((rows, cols), lambda i: (i, 0))


def _onehot(b_ref):
    b = b_ref[0, 0, :]
    ids = jnp.reshape(b, (b.shape[0], 1))
    return (ids == lax.broadcasted_iota(jnp.int32, (b.shape[0], G), 1)).astype(_f32)


def _tc_pre_body(hn_ref, wemb_ref, h0_ref):
    h0_ref[...] = _mm(hn_ref[...], wemb_ref[...])


def _tc_pre(hnp, wemb):
    return _pc(
        _tc_pre_body,
        grid=(GN,),
        in_specs=[_row_spec(CHN, NT), _bcast_spec((NT, ND))],
        out_specs=_row_spec(CHN, ND),
        out_shape=jax.ShapeDtypeStruct((NPAD, ND), _f32),
    )(hnp, wemb)


def _edge_core(he, hs, hd, dfeat, te, w):
    e_in = jnp.concatenate([he, hs, hd, dfeat, te], axis=1)
    pre_e = _mm(e_in, w['ew1']) + w['eb1']
    he_new = he + _mm(jnp.maximum(pre_e, 0.0), w['ew2']) + w['eb2']
    m_in = jnp.concatenate([hs, he_new, dfeat], axis=1)
    pre_m = _mm(m_in, w['mw1']) + w['mb1']
    msg = _mm(jnp.maximum(pre_m, 0.0), w['mw2']) + w['mb2']
    return he_new, msg


def _edge_wdict(refs):
    names = ('ew1', 'eb1', 'ew2', 'eb2', 'mw1', 'mb1', 'mw2', 'mb2')
    return {k: r[...] for k, r in zip(names, refs)}


_EDGE_W_SPECS = [
    _bcast_spec((ED + 2 * ND + NG + 1, ED)), _bcast_spec((1, ED)),
    _bcast_spec((ED, ED)), _bcast_spec((1, ED)),
    _bcast_spec((ND + ED + NG, ND)), _bcast_spec((1, ND)),
    _bcast_spec((ND, ND)), _bcast_spec((1, ND)),
]


def _tc_edge1_body(hein_ref, hs_ref, hd_ref, ps_ref, pd_ref, be_ref, tcol_ref,
                   wemb_ref, *rest):
    wrefs, (he_ref, r_ref, df_ref) = rest[:8], rest[8:]
    he = _mm(hein_ref[...], wemb_ref[...])
    rel = pd_ref[:, 0:3] - ps_ref[:, 0:3]
    dist = jnp.sqrt(jnp.sum(rel * rel, axis=1, keepdims=True) + 1e-8)
    offs = lax.broadcasted_iota(jnp.int32, (1, NG), 1).astype(_f32) * _SMEAR_STEP
    dfeat = jnp.exp(_SMEAR_COEFF * (dist - offs) ** 2)
    df_ref[...] = dfeat
    te = _mm(_onehot(be_ref), tcol_ref[...])
    he_new, msg = _edge_core(he, hs_ref[...], hd_ref[...], dfeat, te,
                             _edge_wdict(wrefs))
    he_ref[...] = he_new
    r_ref[...] = msg


def _tc_edge1(hein, hs, hd, psg, pdg, be3, tcol, wemb, wts):
    return _pc(
        _tc_edge1_body,
        grid=(GE,),
        in_specs=[
            _row_spec(CHE, ET), _row_spec(CHE, ND), _row_spec(CHE, ND),
            _row_spec(CHE, ND), _row_spec(CHE, ND),
            pl.BlockSpec((1, 1, CHE), lambda i: (i, 0, 0)),
            _bcast_spec((G, 1)), _bcast_spec((ET, ED)),
            *_EDGE_W_SPECS,
        ],
        out_specs=[_row_spec(CHE, ED), _row_spec(CHE, ND), _row_spec(CHE, NG)],
        out_shape=[
            jax.ShapeDtypeStruct((E, ED), _f32),
            jax.ShapeDtypeStruct((E, ND), _f32),
            jax.ShapeDtypeStruct((E, NG), _f32),
        ],
    )(hein, hs, hd, psg, pdg, be3, tcol, wemb, *wts)


def _tc_edge23_body(last, he_in_ref, hs_ref, hd_ref, df_ref, be_ref, tcol_ref,
                    *rest):
    if last:
        wrefs, (he_ref, r_ref, esum_ref, ecnt_ref) = rest[:8], rest[8:]
    else:
        wrefs, (he_ref, r_ref) = rest[:8], rest[8:]
    oh = _onehot(be_ref)
    te = _mm(oh, tcol_ref[...])
    he_new, msg = _edge_core(he_in_ref[...], hs_ref[...], hd_ref[...],
                             df_ref[...], te, _edge_wdict(wrefs))
    he_ref[...] = he_new
    r_ref[...] = msg
    if last:
        @pl.when(pl.program_id(0) == 0)
        def _():
            esum_ref[...] = jnp.zeros_like(esum_ref)
            ecnt_ref[...] = jnp.zeros_like(ecnt_ref)

        esum_ref[...] += lax.dot_general(oh, he_new, (((0,), (0,)), ((), ())),
                                         preferred_element_type=_f32)
        ecnt_ref[...] += jnp.sum(oh, axis=0, keepdims=True)


def _tc_edge23(last, he, hs, hd, dfeat, be3, tcol, wts):
    out_specs = [_row_spec(CHE, ED), _row_spec(CHE, ND)]
    out_shape = [jax.ShapeDtypeStruct((E, ED), _f32),
                 jax.ShapeDtypeStruct((E, ND), _f32)]
    if last:
        out_specs += [_bcast_spec((G, ED)), _bcast_spec((1, G))]
        out_shape += [jax.ShapeDtypeStruct((G, ED), _f32),
                      jax.ShapeDtypeStruct((1, G), _f32)]
    return _pc(
        functools.partial(_tc_edge23_body, last),
        grid=(GE,),
        in_specs=[
            _row_spec(CHE, ED), _row_spec(CHE, ND), _row_spec(CHE, ND),
            _row_spec(CHE, NG),
            pl.BlockSpec((1, 1, CHE), lambda i: (i, 0, 0)),
            _bcast_spec((G, 1)),
            *_EDGE_W_SPECS,
        ],
        out_specs=out_specs,
        out_shape=out_shape,
    )(he, hs, hd, dfeat, be3, tcol, *wts)


def _tc_node_body(last, hn_ref, s_ref, bn_ref, tcol_ref, nw1_ref, nb1_ref,
                  nw2_ref, nb2_ref, *rest):
    oh = _onehot(bn_ref)
    nt = _mm(oh, tcol_ref[...])
    n_in = jnp.concatenate([hn_ref[...], s_ref[...], nt], axis=1)
    pre_n = _mm(n_in, nw1_ref[...]) + nb1_ref[...]
    hn_new = (hn_ref[...] + _mm(jnp.maximum(pre_n, 0.0), nw2_ref[...])
              + nb2_ref[...])
    if last:
        hn_out, nsum_ref, ncnt_ref = rest
        hn_out[...] = hn_new

        @pl.when(pl.program_id(0) == 0)
        def _():
            nsum_ref[...] = jnp.zeros_like(nsum_ref)
            ncnt_ref[...] = jnp.zeros_like(ncnt_ref)

        nsum_ref[...] += lax.dot_general(oh, hn_new, (((0,), (0,)), ((), ())),
                                         preferred_element_type=_f32)
        ncnt_ref[...] += jnp.sum(oh, axis=0, keepdims=True)
    else:
        rest[0][...] = hn_new


def _tc_node(last, hn, sacc, bn3, tcol, wts):
    in_specs = [
        _row_spec(CHN, ND), _row_spec(CHN, ND),
        pl.BlockSpec((1, 1, CHN), lambda i: (i, 0, 0)),
        _bcast_spec((G, 1)),
        _bcast_spec((2 * ND + 1, ND)), _bcast_spec((1, ND)),
        _bcast_spec((ND, ND)), _bcast_spec((1, ND)),
    ]
    if last:
        out_specs = [_row_spec(CHN, ND), _bcast_spec((G, ND)),
                     _bcast_spec((1, G))]
        out_shape = [jax.ShapeDtypeStruct((NPAD, ND), _f32),
                     jax.ShapeDtypeStruct((G, ND), _f32),
                     jax.ShapeDtypeStruct((1, G), _f32)]
    else:
        out_specs = [_row_spec(CHN, ND)]
        out_shape = [jax.ShapeDtypeStruct((NPAD, ND), _f32)]
    return _pc(
        functools.partial(_tc_node_body, last),
        grid=(GN,),
        in_specs=in_specs,
        out_specs=out_specs,
        out_shape=out_shape,
    )(hn, sacc, bn3, tcol, *wts)


def _tc_final_body(nsum_ref, ncnt_ref, esum_ref, ecnt_ref,
                   fw1_ref, fb1_ref, fw2_ref, fb2_ref, out_ref):
    cn = jnp.maximum(jnp.transpose(ncnt_ref[...]), 1.0)
    ce = jnp.maximum(jnp.transpose(ecnt_ref[...]), 1.0)
    hsub = jnp.concatenate([nsum_ref[...] / cn, esum_ref[...] / ce], axis=1)
    h = jnp.maximum(_mm(hsub, fw1_ref[...]) + fb1_ref[...], 0.0)
    out_ref[...] = _mm(h, fw2_ref[...]) + fb2_ref[...]


def _tc_final(nsum, ncnt, esum, ecnt, fw1, fb1, fw2, fb2):
    return _pc(
        _tc_final_body,
        out_shape=jax.ShapeDtypeStruct((G, OUT), _f32),
    )(nsum, ncnt, esum, ecnt, fw1, fb1, fw2, fb2)


@functools.lru_cache(maxsize=1)
def _mesh():
    return plsc.VectorSubcoreMesh(core_axis_name="c", subcore_axis_name="s")


def _range32(w):
    per = NGRP // NWORK
    lo = w * per + jnp.minimum(w, NGRP % NWORK)
    hi = lo + per + (w < NGRP % NWORK).astype(jnp.int32)
    return lo, hi


def _range16(s):
    per = NGRP // 16
    lo = s * per + jnp.minimum(s, NGRP % 16)
    hi = lo + per + (s < NGRP % 16).astype(jnp.int32)
    return lo, hi


def _aligned_base(lo):
    base = pl.multiple_of((lo // 8) * 8, 8)
    return base, lo - base


def _drain(dummy_hbm, buf, sem):
    pltpu.make_async_copy(dummy_hbm.at[pl.ds(0, 128)], buf, sem).wait()


def _remap_to_half(idxrow, base):
    for kk in range(8):
        sl = pl.ds(kk * 16, 16)
        v = idxrow[sl] - base
        oob = (v < 0) | (v >= HALF)
        idxrow[sl] = jnp.where(oob, HALF, v)


def _gather_ring(tab_h, gs_h, gd_h, sall, dall, bufs, bufd, gsems, wsems,
                 lo, hi, off):
    def body(j, carry):
        for s in range(NSLOT):
            k = NSLOT * j + s + off
            g = lo + NSLOT * j + s

            @pl.when((j > 0) & (g - NSLOT < hi))
            def _():
                _drain(tab_h, bufs[s], wsems[s])
                _drain(tab_h, bufd[s], wsems[s])

            @pl.when(g < hi)
            def _():
                pltpu.async_copy(tab_h.at[sall.at[k]], bufs[s], gsems[s])
                pltpu.async_copy(tab_h.at[dall.at[k]], bufd[s], gsems[s])

        for s in range(NSLOT):
            k = NSLOT * j + s + off
            g = lo + NSLOT * j + s

            @pl.when(g < hi)
            def _():
                _drain(tab_h, bufs[s], gsems[s])
                _drain(tab_h, bufd[s], gsems[s])
                rows = pl.ds(g * 128, 128)
                pltpu.async_copy(bufs[s], gs_h.at[rows], wsems[s])
                pltpu.async_copy(bufd[s], gd_h.at[rows], wsems[s])

        return carry

    lax.fori_loop(0, NIT32, body, 0)
    for s in range(NSLOT):
        g = lo + NSLOT * (NIT32 - 1) + s

        @pl.when(g < hi)
        def _():
            _drain(tab_h, bufs[s], wsems[s])
            _drain(tab_h, bufd[s], wsems[s])


_GATHER_SCRATCH = [
    pltpu.VMEM((IDXROWS32, 128), jnp.int32),
    pltpu.VMEM((IDXROWS32, 128), jnp.int32),
    pltpu.VMEM((NSLOT, 128, ND), _f32),
    pltpu.VMEM((NSLOT, 128, ND), _f32),
] + [pltpu.SemaphoreType.DMA] * (2 * NSLOT)


def _sc_gather_h(h, src2, dst2):
    @functools.partial(
        pl.kernel, mesh=_mesh(),
        out_type=[jax.ShapeDtypeStruct((E, ND), _f32),
                  jax.ShapeDtypeStruct((E, ND), _f32)],
        scratch_types=_GATHER_SCRATCH,
    )
    def k(h_h, src_h, dst_h, hs_h, hd_h, sall, dall, bufs, bufd, *sems):
        gsems, wsems = sems[:NSLOT], sems[NSLOT:]
        w = lax.axis_index("s") * 2 + lax.axis_index("c")
        lo, hi = _range32(w)
        p8, off = _aligned_base(lo)
        pltpu.sync_copy(src_h.at[pl.ds(p8, IDXROWS32)], sall)
        pltpu.sync_copy(dst_h.at[pl.ds(p8, IDXROWS32)], dall)
        bs = [bufs.at[s] for s in range(NSLOT)]
        bd = [bufd.at[s] for s in range(NSLOT)]
        _gather_ring(h_h, hs_h, hd_h, sall, dall, bs, bd, gsems, wsems,
                     lo, hi, off)

    return k(h, src2, dst2)


def _sc_scatter(r, dst2, znd):
    @functools.partial(
        pl.kernel, mesh=_mesh(),
        out_type=jax.ShapeDtypeStruct((NPAD, ND), _f32),
        scratch_types=[pltpu.VMEM((IDXROWS16, 128), jnp.int32),
                       pltpu.VMEM((NSLOT, 128, ND), _f32),
                       pltpu.VMEM_SHARED((SPAD, ND), _f32)]
        + [pltpu.SemaphoreType.DMA] * NSLOT,
    )
    def k(r_h, dst_h, znd_h, s_h, dall, rbuf, s_sh, *rsems):
        cid = lax.axis_index("c")
        s = lax.axis_index("s")
        base = cid * HALF
        lo, hi = _range16(s)
        p8, off = _aligned_base(lo)
        my = pl.ds(s * RPS, RPS)
        pltpu.sync_copy(znd_h.at[my], s_sh.at[my])
        pltpu.sync_copy(dst_h.at[pl.ds(p8, IDXROWS16)], dall)
        plsc.subcore_barrier()

        @pl.when(lo < hi)
        def _():
            pltpu.async_copy(r_h.at[pl.ds(lo * 128, 128)], rbuf.at[0],
                             rsems[0])

        def body(j, carry):
            for sl in range(NSLOT):
                k_ = NSLOT * j + sl
                g = lo + k_
                bp = (sl + 1) % NSLOT
                kn = k_ + 1
                gn = lo + kn

                @pl.when(gn < hi)
                def _():
                    pltpu.async_copy(r_h.at[pl.ds(gn * 128, 128)],
                                     rbuf.at[bp], rsems[bp])

                @pl.when(g < hi)
                def _():
                    _drain(r_h, rbuf.at[sl], rsems[sl])
                    _remap_to_half(dall.at[k_ + off], base)
                    pltpu.sync_copy(rbuf.at[sl], s_sh.at[dall.at[k_ + off]],
                                    add=True)

            return carry

        lax.fori_loop(0, NIT16, body, 0)
        plsc.subcore_barrier()
        pltpu.sync_copy(s_sh.at[my], s_h.at[pl.ds(base + s * RPS, RPS)])

    return k(r, dst2, znd)


def _block_weights(blk):
    edge = (blk['edge_W1'], blk['edge_b1'].reshape(1, ED), blk['edge_W2'],
            blk['edge_b2'].reshape(1, ED), blk['msg_W1'],
            blk['msg_b1'].reshape(1, ND), blk['msg_W2'],
            blk['msg_b2'].reshape(1, ND))
    node = (blk['node_W1'], blk['node_b1'].reshape(1, ND), blk['node_W2'],
            blk['node_b2'].reshape(1, ND))
    return edge, node


def kernel(h_node, pos_node, batch_node, h_edge, edge_index, batch_edge, t,
           params):
    blocks = params['blocks']
    hnp = jnp.pad(h_node, ((0, NPAD - N), (0, 0)))
    posp = jnp.pad(pos_node, ((0, NPAD - N), (0, ND - 3)))
    bn3 = jnp.pad(batch_node, (0, NPAD - N), constant_values=G).reshape(
        GN, 1, CHN)
    be3 = batch_edge.reshape(GE, 1, CHE)
    src2 = jnp.pad(edge_index[0].reshape(NGRP, 128),
                   ((0, NGRPP - NGRP), (0, 0)))
    dst2 = jnp.pad(edge_index[1].reshape(NGRP, 128),
                   ((0, NGRPP - NGRP), (0, 0)))
    tcol = t.astype(_f32).reshape(G, 1)
    znd = jnp.zeros((NPAD, ND), _f32)

    ew, nw = zip(*(_block_weights(b) for b in blocks))

    h0 = _tc_pre(hnp, params['W_node_emb'])
    psg, pdg = _sc_gather_h(posp, src2, dst2)
    hs, hd = _sc_gather_h(h0, src2, dst2)
    he, r, dfeat = _tc_edge1(h_edge, hs, hd, psg, pdg, be3, tcol,
                             params['W_edge_emb'], ew[0])
    sacc = _sc_scatter(r, dst2, znd)
    hn = _tc_node(False, h0, sacc, bn3, tcol, nw[0])[0]

    hs, hd = _sc_gather_h(hn, src2, dst2)
    he, r = _tc_edge23(False, he, hs, hd, dfeat, be3, tcol, ew[1])
    sacc = _sc_scatter(r, dst2, znd)
    hn = _tc_node(False, hn, sacc, bn3, tcol, nw[1])[0]

    hs, hd = _sc_gather_h(hn, src2, dst2)
    he, r, esum, ecnt = _tc_edge23(True, he, hs, hd, dfeat, be3, tcol, ew[2])
    sacc = _sc_scatter(r, dst2, znd)
    hn, nsum, ncnt = _tc_node(True, hn, sacc, bn3, tcol, nw[2])

    return _tc_final(nsum, ncnt, esum, ecnt, params['final_W1'],
                     params['final_b1'].reshape(1, ND + ED),
                     params['final_W2'], params['final_b2'].reshape(1, OUT))
